# baseline (speedup 1.0000x reference)
import functools

import numpy as np
import jax
import jax.numpy as jnp
from jax import lax
from jax.experimental import pallas as pl
from jax.experimental.pallas import tpu as pltpu
from jax.experimental.pallas import tpu_sc as plsc

_N_NODES = 100000
_N_EDGES = 3200000
_D_EDGE = 16
_K = 16
_R_MIN = 0.0
_R_MAX = 4.0

_NC = 2
_NS = 16
_NW = _NC * _NS
_NT = _N_EDGES // 128
_T = 8
_BE = _T * 128
_NGRP = _BE // 16
_BPW = -(-_NT // (_NW * _T))
_SPAN = _BPW * _T
_ATTR_W = _N_EDGES * _D_EDGE // _NW

_DELTA = (_R_MAX - _R_MIN) / _K
_GAMMA = np.float32(1.0 / (2.0 * _DELTA ** 2 + 1e-09))


def _rsqrt(x):
    bi = lax.bitcast_convert_type(x, jnp.int32)
    yi = jnp.int32(0x5F3759DF) - lax.shift_right_logical(bi, 1)
    y = lax.bitcast_convert_type(yi, jnp.float32)
    h = x * jnp.float32(0.5)
    for _ in range(3):
        y = y * (jnp.float32(1.5) - h * y * y)
    return y


def _body(pos_hbm, ei_hbm, attr_hbm, out_hbm,
          idx_v, ex_v, ey_v, ez_v, fx_v, fy_v, fz_v,
          rx_v, ry_v, rz_v, cx_v, cy_v, cz_v, out_v,
          pf_sh, isem, gsem, osem, absem):
    c = lax.axis_index("c")
    s = lax.axis_index("s")
    wid = s * _NC + c
    t0 = jnp.minimum(wid * _SPAN, _NT - _SPAN)

    @pl.when(s == 0)
    def _stage():
        pltpu.sync_copy(pos_hbm, pf_sh)

    plsc.subcore_barrier()

    attr_cp = pltpu.async_copy(
        attr_hbm.at[pl.ds(wid * _ATTR_W, _ATTR_W)],
        out_hbm.at[pl.ds(wid * _ATTR_W, _ATTR_W)], absem)

    neg_gamma = jnp.float32(-_GAMMA)
    kstep = jnp.float32((_R_MAX - _R_MIN) / (_K - 1))

    coord = [(rx_v[0], ry_v[0], rz_v[0], cx_v[0], cy_v[0], cz_v[0]),
             (rx_v[1], ry_v[1], rz_v[1], cx_v[1], cy_v[1], cz_v[1])]
    gidx = [(ex_v[0], ey_v[0], ez_v[0], fx_v[0], fy_v[0], fz_v[0]),
            (ex_v[1], ey_v[1], ez_v[1], fx_v[1], fy_v[1], fz_v[1])]

    def fire_idx(b, p):
        pltpu.async_copy(
            ei_hbm.at[pl.ds((t0 + b * _T) * 256, _T * 256)], idx_v[p],
            isem[p])

    def drain_idx(p):
        pltpu.make_async_copy(
            ei_hbm.at[pl.ds(0, _T * 256)], idx_v[p], isem[p]).wait()

    def transform_idx(p):
        ex, ey, ez, fx, fy, fz = gidx[p]
        iv = idx_v[p]

        def tf_body(g, carry2):
            base = (g >> 3) * 256 + (g & 7) * 16
            sl = pl.ds(g * 16, 16)
            r3 = iv[pl.ds(base, 16)] * 3
            c3 = iv[pl.ds(base + 128, 16)] * 3
            ex[sl] = r3
            ey[sl] = r3 + 1
            ez[sl] = r3 + 2
            fx[sl] = c3
            fy[sl] = c3 + 1
            fz[sl] = c3 + 2
            return carry2

        lax.fori_loop(0, _NGRP, tf_body, 0)

    def fire_gathers(p):
        for t in range(6):
            pltpu.async_copy(pf_sh.at[gidx[p][t]], coord[p][t], gsem[p])

    def drain_gathers(p):
        for t in range(6):
            pltpu.make_async_copy(
                pf_sh.at[gidx[p][t]], coord[p][t], gsem[p]).wait()

    def fire_out(b, p):
        tb = t0 + b * _T
        pltpu.async_copy(
            out_v[p].at[pl.ds(0, _T * 1024)],
            out_hbm.at[pl.ds((2 * _NT + tb) * 1024, _T * 1024)], osem[p])
        pltpu.async_copy(
            out_v[p].at[pl.ds(_T * 1024, _T * 1024)],
            out_hbm.at[pl.ds((3 * _NT + tb) * 1024, _T * 1024)], osem[p])

    def drain_out(p):
        pltpu.make_async_copy(
            out_v[p].at[pl.ds(0, _T * 1024)],
            out_hbm.at[pl.ds(0, _T * 1024)], osem[p]).wait()
        pltpu.make_async_copy(
            out_v[p].at[pl.ds(_T * 1024, _T * 1024)],
            out_hbm.at[pl.ds(0, _T * 1024)], osem[p]).wait()

    def compute(p):
        rx, ry, rz, cx, cy, cz = coord[p]
        ov = out_v[p]

        def grp_body(g, carry2):
            sl = pl.ds(g * 16, 16)
            dx = rx[sl] - cx[sl]
            dy = ry[sl] - cy[sl]
            dz = rz[sl] - cz[sl]
            d2 = dx * dx + dy * dy + dz * dz
            dist = d2 * _rsqrt(d2)
            obase = (g >> 3) * 1024 + (g & 7) * 16
            for j in range(_K):
                t = dist - jnp.float32(j) * kstep
                off = obase + (j // 8) * (_T * 1024) + (j % 8) * 128
                ov[pl.ds(off, 16)] = jnp.exp(t * t * neg_gamma)
            return carry2

        lax.fori_loop(0, _NGRP, grp_body, 0)

    def half_step(b, p, with_out_drain):
        q = 1 - p
        nxt2 = jnp.minimum(b + 2, _BPW - 1)
        drain_gathers(p)
        drain_idx(q)
        transform_idx(q)
        fire_gathers(q)
        fire_idx(nxt2, p)
        if with_out_drain:
            drain_out(p)
        compute(p)
        fire_out(b, p)

    pltpu.sync_copy(ei_hbm.at[pl.ds(t0 * 256, _T * 256)], idx_v[0])
    transform_idx(0)
    fire_gathers(0)
    fire_idx(1, 1)

    half_step(jnp.int32(0), 0, False)
    half_step(jnp.int32(1), 1, False)

    def pair_body(i, carry):
        b = i * 2
        half_step(b, 0, True)
        half_step(b + 1, 1, True)
        return carry

    lax.fori_loop(1, _BPW // 2, pair_body, 0)

    drain_gathers(0)
    drain_idx(1)
    drain_out(1)
    drain_out(0)
    attr_cp.wait()


@jax.jit
def kernel(pos, edge_index, edge_attr):
    mesh = plsc.VectorSubcoreMesh(core_axis_name="c", subcore_axis_name="s")
    ivec = pltpu.VMEM((_BE,), jnp.int32)
    fvec = pltpu.VMEM((_BE,), jnp.float32)
    f = pl.kernel(
        _body,
        out_type=jax.ShapeDtypeStruct((_N_EDGES * (_D_EDGE + _K),),
                                      jnp.float32),
        mesh=mesh,
        scratch_types=[
            (pltpu.VMEM((_T * 256,), jnp.int32),
             pltpu.VMEM((_T * 256,), jnp.int32)),
            (ivec, ivec), (ivec, ivec), (ivec, ivec),
            (ivec, ivec), (ivec, ivec), (ivec, ivec),
            (fvec, fvec), (fvec, fvec), (fvec, fvec),
            (fvec, fvec), (fvec, fvec), (fvec, fvec),
            (pltpu.VMEM((2 * _T * 1024,), jnp.float32),
             pltpu.VMEM((2 * _T * 1024,), jnp.float32)),
            pltpu.VMEM_SHARED((3 * _N_NODES,), jnp.float32),
            (pltpu.SemaphoreType.DMA, pltpu.SemaphoreType.DMA),
            (pltpu.SemaphoreType.DMA, pltpu.SemaphoreType.DMA),
            (pltpu.SemaphoreType.DMA, pltpu.SemaphoreType.DMA),
            pltpu.SemaphoreType.DMA,
        ],
        compiler_params=pltpu.CompilerParams(use_tc_tiling_on_sc=False),
    )
    ei_phys = edge_index.reshape(2, _NT, 128).transpose(1, 0, 2).reshape(-1)
    attr_phys = edge_attr.reshape(_NT, 128, _D_EDGE // 8, 8).transpose(
        2, 0, 3, 1).reshape(-1)
    raw = f(pos.reshape(-1), ei_phys, attr_phys)
    out = raw.reshape(4, _NT, 8, 128).transpose(1, 3, 0, 2).reshape(
        _N_EDGES, _D_EDGE + _K)
    return out


# speedup vs baseline: 13.6683x; 13.6683x over previous
import functools

import numpy as np
import jax
import jax.numpy as jnp
from jax import lax
from jax.experimental import pallas as pl
from jax.experimental.pallas import tpu as pltpu
from jax.experimental.pallas import tpu_sc as plsc

_N_NODES = 100000
_N_EDGES = 3200000
_D_EDGE = 16
_K = 16
_R_MIN = 0.0
_R_MAX = 4.0

_NC = 2
_NS = 16
_NW = _NC * _NS
_NT = _N_EDGES // 128
_T = 8
_BE = _T * 128
_NGRP = _BE // 16
_BPW = -(-_NT // (_NW * _T))
_SPAN = _BPW * _T
_ATTR_W = _N_EDGES * _D_EDGE // _NW

_DELTA = (_R_MAX - _R_MIN) / _K
_GAMMA = np.float32(1.0 / (2.0 * _DELTA ** 2 + 1e-09))


def _rsqrt(x):
    bi = lax.bitcast_convert_type(x, jnp.int32)
    yi = jnp.int32(0x5F3759DF) - lax.shift_right_logical(bi, 1)
    y = lax.bitcast_convert_type(yi, jnp.float32)
    h = x * jnp.float32(0.5)
    for _ in range(3):
        y = y * (jnp.float32(1.5) - h * y * y)
    return y


def _body(pos_hbm, ei_hbm, attr_hbm, out_hbm,
          idx_v, ex_v, ey_v, ez_v, fx_v, fy_v, fz_v,
          rx_v, ry_v, rz_v, cx_v, cy_v, cz_v, out_v, ab_v,
          pf_sh, isem, gsem, osem, arsem, awsem):
    c = lax.axis_index("c")
    s = lax.axis_index("s")
    wid = s * _NC + c
    t0 = jnp.minimum(wid * _SPAN, _NT - _SPAN)

    @pl.when(s == 0)
    def _stage():
        pltpu.sync_copy(pos_hbm, pf_sh)

    plsc.subcore_barrier()

    _ACH = 16384
    alast = wid * _ATTR_W + (_ATTR_W - _ACH)

    def attr_off(b):
        return jnp.minimum(wid * _ATTR_W + b * _ACH, alast)

    def fire_aread(b, p):
        pltpu.async_copy(attr_hbm.at[pl.ds(attr_off(b), _ACH)], ab_v[p],
                         arsem[p])

    def drain_aread(p):
        pltpu.make_async_copy(attr_hbm.at[pl.ds(0, _ACH)], ab_v[p],
                              arsem[p]).wait()

    def fire_awrite(b, p):
        pltpu.async_copy(ab_v[p], out_hbm.at[pl.ds(attr_off(b), _ACH)],
                         awsem[p])

    def drain_awrite(p):
        pltpu.make_async_copy(ab_v[p], out_hbm.at[pl.ds(0, _ACH)],
                              awsem[p]).wait()

    neg_gamma = jnp.float32(-_GAMMA)
    kstep = jnp.float32((_R_MAX - _R_MIN) / (_K - 1))

    coord = [(rx_v[0], ry_v[0], rz_v[0], cx_v[0], cy_v[0], cz_v[0]),
             (rx_v[1], ry_v[1], rz_v[1], cx_v[1], cy_v[1], cz_v[1])]
    gidx = [(ex_v[0], ey_v[0], ez_v[0], fx_v[0], fy_v[0], fz_v[0]),
            (ex_v[1], ey_v[1], ez_v[1], fx_v[1], fy_v[1], fz_v[1])]

    def fire_idx(b, p):
        pltpu.async_copy(
            ei_hbm.at[pl.ds((t0 + b * _T) * 256, _T * 256)], idx_v[p],
            isem[p])

    def drain_idx(p):
        pltpu.make_async_copy(
            ei_hbm.at[pl.ds(0, _T * 256)], idx_v[p], isem[p]).wait()

    def transform_idx(p):
        ex, ey, ez, fx, fy, fz = gidx[p]
        iv = idx_v[p]

        def tf_body(g, carry2):
            base = (g >> 3) * 256 + (g & 7) * 16
            sl = pl.ds(g * 16, 16)
            r3 = iv[pl.ds(base, 16)] * 3
            c3 = iv[pl.ds(base + 128, 16)] * 3
            ex[sl] = r3
            ey[sl] = r3 + 1
            ez[sl] = r3 + 2
            fx[sl] = c3
            fy[sl] = c3 + 1
            fz[sl] = c3 + 2
            return carry2

        lax.fori_loop(0, _NGRP, tf_body, 0)

    def fire_gathers(p):
        for t in range(6):
            pltpu.async_copy(pf_sh.at[gidx[p][t]], coord[p][t], gsem[p])

    def drain_gathers(p):
        for t in range(6):
            pltpu.make_async_copy(
                pf_sh.at[gidx[p][t]], coord[p][t], gsem[p]).wait()

    def fire_out(b, p):
        tb = t0 + b * _T
        pltpu.async_copy(
            out_v[p].at[pl.ds(0, _T * 1024)],
            out_hbm.at[pl.ds((2 * _NT + tb) * 1024, _T * 1024)], osem[p])
        pltpu.async_copy(
            out_v[p].at[pl.ds(_T * 1024, _T * 1024)],
            out_hbm.at[pl.ds((3 * _NT + tb) * 1024, _T * 1024)], osem[p])

    def drain_out(p):
        pltpu.make_async_copy(
            out_v[p].at[pl.ds(0, _T * 1024)],
            out_hbm.at[pl.ds(0, _T * 1024)], osem[p]).wait()
        pltpu.make_async_copy(
            out_v[p].at[pl.ds(_T * 1024, _T * 1024)],
            out_hbm.at[pl.ds(0, _T * 1024)], osem[p]).wait()

    def compute(p):
        rx, ry, rz, cx, cy, cz = coord[p]
        ov = out_v[p]

        def grp_body(g, carry2):
            sl = pl.ds(g * 16, 16)
            dx = rx[sl] - cx[sl]
            dy = ry[sl] - cy[sl]
            dz = rz[sl] - cz[sl]
            d2 = dx * dx + dy * dy + dz * dz
            dist = d2 * _rsqrt(d2)
            obase = (g >> 3) * 1024 + (g & 7) * 16
            for j in range(_K):
                t = dist - jnp.float32(j) * kstep
                off = obase + (j // 8) * (_T * 1024) + (j % 8) * 128
                ov[pl.ds(off, 16)] = jnp.exp(t * t * neg_gamma)
            return carry2

        lax.fori_loop(0, _NGRP, grp_body, 0)

    def half_step(b, p, with_out_drain):
        q = 1 - p
        nxt2 = jnp.minimum(b + 2, _BPW - 1)
        drain_gathers(p)
        drain_idx(q)
        if with_out_drain:
            drain_awrite(p)
        fire_aread(b, p)
        transform_idx(q)
        fire_gathers(q)
        fire_idx(nxt2, p)
        if with_out_drain:
            drain_out(p)
        compute(p)
        drain_aread(p)
        fire_awrite(b, p)
        fire_out(b, p)

    pltpu.sync_copy(ei_hbm.at[pl.ds(t0 * 256, _T * 256)], idx_v[0])
    transform_idx(0)
    fire_gathers(0)
    fire_idx(1, 1)

    half_step(jnp.int32(0), 0, False)
    half_step(jnp.int32(1), 1, False)

    def pair_body(i, carry):
        b = i * 2
        half_step(b, 0, True)
        half_step(b + 1, 1, True)
        return carry

    lax.fori_loop(1, _BPW // 2, pair_body, 0)

    drain_gathers(0)
    drain_idx(1)
    drain_out(1)
    drain_out(0)
    drain_awrite(0)
    drain_awrite(1)


@jax.jit
def kernel(pos, edge_index, edge_attr):
    mesh = plsc.VectorSubcoreMesh(core_axis_name="c", subcore_axis_name="s")
    ivec = pltpu.VMEM((_BE,), jnp.int32)
    fvec = pltpu.VMEM((_BE,), jnp.float32)
    f = pl.kernel(
        _body,
        out_type=jax.ShapeDtypeStruct((_N_EDGES * (_D_EDGE + _K),),
                                      jnp.float32),
        mesh=mesh,
        scratch_types=[
            (pltpu.VMEM((_T * 256,), jnp.int32),
             pltpu.VMEM((_T * 256,), jnp.int32)),
            (ivec, ivec), (ivec, ivec), (ivec, ivec),
            (ivec, ivec), (ivec, ivec), (ivec, ivec),
            (fvec, fvec), (fvec, fvec), (fvec, fvec),
            (fvec, fvec), (fvec, fvec), (fvec, fvec),
            (pltpu.VMEM((2 * _T * 1024,), jnp.float32),
             pltpu.VMEM((2 * _T * 1024,), jnp.float32)),
            (pltpu.VMEM((16384,), jnp.float32),
             pltpu.VMEM((16384,), jnp.float32)),
            pltpu.VMEM_SHARED((3 * _N_NODES,), jnp.float32),
            (pltpu.SemaphoreType.DMA, pltpu.SemaphoreType.DMA),
            (pltpu.SemaphoreType.DMA, pltpu.SemaphoreType.DMA),
            (pltpu.SemaphoreType.DMA, pltpu.SemaphoreType.DMA),
            (pltpu.SemaphoreType.DMA, pltpu.SemaphoreType.DMA),
            (pltpu.SemaphoreType.DMA, pltpu.SemaphoreType.DMA),
        ],
        compiler_params=pltpu.CompilerParams(use_tc_tiling_on_sc=False),
    )
    ei_phys = edge_index.reshape(2, _NT, 128).transpose(1, 0, 2).reshape(-1)
    attr_phys = edge_attr.reshape(_NT, 128, _D_EDGE // 8, 8).transpose(
        2, 0, 3, 1).reshape(-1)
    raw = f(pos.reshape(-1), ei_phys, attr_phys)
    out = raw.reshape(4, _NT, 8, 128).transpose(1, 3, 0, 2).reshape(
        _N_EDGES, _D_EDGE + _K)
    return out
